# Initial kernel scaffold; baseline (speedup 1.0000x reference)
import jax
import jax.numpy as jnp
from jax.experimental import pallas as pl


def kernel(x1, x2):
    raise NotImplementedError("write your pallas kernel here")


# speedup vs baseline: 33.1007x; 33.1007x over previous
import jax
import jax.numpy as jnp
from jax import lax
from jax.experimental import pallas as pl

_N = 4096
_R = 512
_NN = 6
_KS = ((1, -1, 100), (3, 2, 20), (5, 4, 10))


def _knn_kernel(rows_ref, cols_ref, out_ref):
    rx = rows_ref[0, :, 0:1]
    ry = rows_ref[0, :, 1:2]
    cx = cols_ref[0, 0:1, :]
    cy = cols_ref[0, 1:2, :]
    d2 = (rx - cx) ** 2 + (ry - cy) ** 2
    iota = lax.broadcasted_iota(jnp.int32, (_R, _N), 1)
    for k in range(_NN):
        m = jnp.min(d2, axis=1, keepdims=True)
        out_ref[0, :, k:k + 1] = jnp.sqrt(m + 1e-12)
        if k + 1 < _NN:
            idx = jnp.min(jnp.where(d2 == m, iota, _N), axis=1, keepdims=True)
            d2 = jnp.where(iota == idx, jnp.inf, d2)


def _loss_kernel(nn_ref, out_ref):
    iota = (lax.broadcasted_iota(jnp.int32, (32, 128), 0) * 128
            + lax.broadcasted_iota(jnp.int32, (32, 128), 1))

    def lengths(s, hi, lo):
        v = nn_ref[s, :, hi].reshape(32, 128)
        if lo >= 0:
            v = v - nn_ref[s, :, lo].reshape(32, 128)
        return v

    loss = jnp.float32(0.0)
    for hi, lo, k in _KS:
        a = lengths(0, hi, lo)
        b = lengths(1, hi, lo)

        def body(_, carry):
            a, b, acc = carry
            ma = jnp.max(a)
            ia = jnp.min(jnp.where(a == ma, iota, _N))
            a = jnp.where(iota == ia, -jnp.inf, a)
            mb = jnp.max(b)
            ib = jnp.min(jnp.where(b == mb, iota, _N))
            b = jnp.where(iota == ib, -jnp.inf, b)
            return a, b, acc + (ma - mb) ** 2

        _, _, acc = lax.fori_loop(0, k, body, (a, b, jnp.float32(0.0)))
        loss = loss + acc / k
    out_ref[...] = jnp.full((8, 128), loss, dtype=jnp.float32)


def kernel(x1, x2):
    c, h, w = x1.shape
    p1 = jnp.transpose(x1, (1, 2, 0)).reshape(-1, c)
    p2 = jnp.transpose(x2, (1, 2, 0)).reshape(-1, c)
    rows = jnp.stack([p1, p2])
    cols = jnp.stack([p1.T, p2.T])

    nn = pl.pallas_call(
        _knn_kernel,
        grid=(2, _N // _R),
        in_specs=[
            pl.BlockSpec((1, _R, 2), lambda s, i: (s, i, 0)),
            pl.BlockSpec((1, 2, _N), lambda s, i: (s, 0, 0)),
        ],
        out_specs=pl.BlockSpec((1, _R, 8), lambda s, i: (s, i, 0)),
        out_shape=jax.ShapeDtypeStruct((2, _N, 8), jnp.float32),
    )(rows, cols)

    loss = pl.pallas_call(
        _loss_kernel,
        out_shape=jax.ShapeDtypeStruct((8, 128), jnp.float32),
    )(nn)
    return loss[0, 0]


# speedup vs baseline: 36.0926x; 1.0904x over previous
import jax
import jax.numpy as jnp
from jax import lax
from jax.experimental import pallas as pl
from jax.experimental.pallas import tpu as pltpu

_N = 4096
_R = 512
_NN = 6
_KS = ((1, -1, 100), (3, 2, 20), (5, 4, 10))


def _knn_kernel(rows_ref, cols_ref, out_ref):
    i = pl.program_id(1)
    rx = rows_ref[0, :, 0:1]
    ry = rows_ref[0, :, 1:2]
    cx = cols_ref[0, 0:1, :]
    cy = cols_ref[0, 1:2, :]
    d2 = (rx - cx) ** 2 + (ry - cy) ** 2
    iota = lax.broadcasted_iota(jnp.int32, (_R, _N), 1)
    self_col = lax.broadcasted_iota(jnp.int32, (_R, _N), 0) + i * _R
    d2 = jnp.where(iota == self_col, jnp.inf, d2)
    for k in range(1, _NN):
        m = jnp.min(d2, axis=1, keepdims=True)
        out_ref[0, :, k:k + 1] = jnp.sqrt(m + 1e-12)
        if k + 1 < _NN:
            idx = jnp.min(jnp.where(d2 == m, iota, _N), axis=1, keepdims=True)
            d2 = jnp.where(iota == idx, jnp.inf, d2)


def _loss_kernel(nn_ref, out_ref):
    iota = (lax.broadcasted_iota(jnp.int32, (32, 128), 0) * 128
            + lax.broadcasted_iota(jnp.int32, (32, 128), 1))

    def lengths(s, hi, lo):
        v = nn_ref[s, :, hi].reshape(32, 128)
        if lo >= 0:
            v = v - nn_ref[s, :, lo].reshape(32, 128)
        return v

    loss = jnp.float32(0.0)
    for hi, lo, k in _KS:
        a = lengths(0, hi, lo)
        b = lengths(1, hi, lo)

        def body(_, carry):
            a, b, acc = carry
            ma = jnp.max(a)
            ia = jnp.min(jnp.where(a == ma, iota, _N))
            a = jnp.where(iota == ia, -jnp.inf, a)
            mb = jnp.max(b)
            ib = jnp.min(jnp.where(b == mb, iota, _N))
            b = jnp.where(iota == ib, -jnp.inf, b)
            return a, b, acc + (ma - mb) ** 2

        _, _, acc = lax.fori_loop(0, k, body, (a, b, jnp.float32(0.0)))
        loss = loss + acc / k
    out_ref[...] = jnp.full((8, 128), loss, dtype=jnp.float32)


def kernel(x1, x2):
    c, h, w = x1.shape
    p1 = jnp.transpose(x1, (1, 2, 0)).reshape(-1, c)
    p2 = jnp.transpose(x2, (1, 2, 0)).reshape(-1, c)
    rows = jnp.stack([p1, p2])
    cols = jnp.stack([p1.T, p2.T])

    nn = pl.pallas_call(
        _knn_kernel,
        grid=(2, _N // _R),
        in_specs=[
            pl.BlockSpec((1, _R, 2), lambda s, i: (s, i, 0)),
            pl.BlockSpec((1, 2, _N), lambda s, i: (s, 0, 0)),
        ],
        out_specs=pl.BlockSpec((1, _R, 8), lambda s, i: (s, i, 0)),
        out_shape=jax.ShapeDtypeStruct((2, _N, 8), jnp.float32),
        compiler_params=pltpu.CompilerParams(
            dimension_semantics=("parallel", "parallel")),
    )(rows, cols)

    loss = pl.pallas_call(
        _loss_kernel,
        out_shape=jax.ShapeDtypeStruct((8, 128), jnp.float32),
    )(nn)
    return loss[0, 0]


# speedup vs baseline: 47.0816x; 1.3045x over previous
import functools

import jax
import jax.numpy as jnp
from jax import lax
from jax.experimental import pallas as pl
from jax.experimental.pallas import tpu as pltpu
from jax.experimental.shard_map import shard_map
from jax.sharding import PartitionSpec as P

_N = 4096
_R = 512
_NN = 6
_KS = ((1, -1, 100), (3, 2, 20), (5, 4, 10))


def _knn_kernel(rows_ref, cols_ref, out_ref):
    i = pl.program_id(1)
    rx = rows_ref[0:1, :, 0:1].reshape(_R, 1)
    ry = rows_ref[0:1, :, 1:2].reshape(_R, 1)
    cx = cols_ref[0:1, 0:1, :].reshape(1, _N)
    cy = cols_ref[0:1, 1:2, :].reshape(1, _N)
    d2 = (rx - cx) ** 2 + (ry - cy) ** 2
    iota = lax.broadcasted_iota(jnp.int32, (_R, _N), 1)
    self_col = lax.broadcasted_iota(jnp.int32, (_R, _N), 0) + i * _R
    d2 = jnp.where(iota == self_col, jnp.inf, d2)

    m = [jnp.full((_R, 128), jnp.inf, dtype=jnp.float32) for _ in range(_NN)]
    for c in range(_N // 128):
        new = d2[:, c * 128:(c + 1) * 128]
        for j in range(_NN):
            lo = jnp.minimum(m[j], new)
            new = jnp.maximum(m[j], new)
            m[j] = lo

    cat = jnp.concatenate(m, axis=1)
    w = cat.shape[1]
    miota = lax.broadcasted_iota(jnp.int32, (_R, w), 1)
    for k in range(1, _NN):
        mn = jnp.min(cat, axis=1, keepdims=True)
        out_ref[0:1, :, k:k + 1] = jnp.sqrt(mn + 1e-12).reshape(1, _R, 1)
        if k + 1 < _NN:
            idx = jnp.min(jnp.where(cat == mn, miota, w), axis=1, keepdims=True)
            cat = jnp.where(miota == idx, jnp.inf, cat)


def _bitonic_sort_asc(x, flat):
    for k in range(1, 13):
        kbit = 1 << k
        asc = (flat & kbit) == 0
        for j in range(k - 1, -1, -1):
            s = 1 << j
            if s < 128:
                pm = pltpu.roll(x, 128 - s, 2)
                pp = pltpu.roll(x, s, 2)
            else:
                r = s // 128
                pm = pltpu.roll(x, 32 - r, 1)
                pp = pltpu.roll(x, r, 1)
            low = (flat & s) == 0
            p = jnp.where(low[None], pm, pp)
            keep_min = (low == asc)[None]
            x = jnp.where(keep_min, jnp.minimum(x, p), jnp.maximum(x, p))
    return x


def _sort_kernel(nn_ref, out_ref):
    flat = (lax.broadcasted_iota(jnp.int32, (32, 128), 0) * 128
            + lax.broadcasted_iota(jnp.int32, (32, 128), 1))

    def lengths(hi, lo):
        v = nn_ref[0:1, :, hi:hi + 1].reshape(32, 128)
        if lo >= 0:
            v = v - nn_ref[0:1, :, lo:lo + 1].reshape(32, 128)
        return v

    arrs = [lengths(hi, lo) for hi, lo, _ in _KS]
    out_ref[...] = _bitonic_sort_asc(jnp.stack(arrs), flat).reshape(1, 3, 32, 128)


def _mse_kernel(s_ref, out_ref):
    flat = (lax.broadcasted_iota(jnp.int32, (32, 128), 0) * 128
            + lax.broadcasted_iota(jnp.int32, (32, 128), 1))
    loss = jnp.float32(0.0)
    for i, (hi, lo, k) in enumerate(_KS):
        w = jnp.where(flat >= _N - k, jnp.float32(1.0 / k), jnp.float32(0.0))
        d = (s_ref[0:1, i:i + 1].reshape(32, 128)
             - s_ref[1:2, i:i + 1].reshape(32, 128))
        loss = loss + jnp.sum(d * d * w)
    out_ref[...] = jnp.full((8, 128), loss, dtype=jnp.float32)


def _per_set(rows_l, cols_l):
    nn = pl.pallas_call(
        _knn_kernel,
        grid=(1, _N // _R),
        in_specs=[
            pl.BlockSpec((1, _R, 2), lambda s, i: (s, i, 0)),
            pl.BlockSpec((1, 2, _N), lambda s, i: (s, 0, 0)),
        ],
        out_specs=pl.BlockSpec((1, _R, 8), lambda s, i: (s, i, 0)),
        out_shape=jax.ShapeDtypeStruct((1, _N, 8), jnp.float32),
        compiler_params=pltpu.CompilerParams(
            dimension_semantics=("parallel", "parallel")),
    )(rows_l, cols_l)

    return pl.pallas_call(
        _sort_kernel,
        out_shape=jax.ShapeDtypeStruct((1, 3, 32, 128), jnp.float32),
    )(nn)


def kernel(x1, x2):
    c, h, w = x1.shape
    p1 = jnp.transpose(x1, (1, 2, 0)).reshape(-1, c)
    p2 = jnp.transpose(x2, (1, 2, 0)).reshape(-1, c)
    rows = jnp.stack([p1, p2])
    cols = jnp.stack([p1.T, p2.T])

    mesh = jax.make_mesh((2,), ("s",),
                         axis_types=(jax.sharding.AxisType.Auto,))
    sharded = shard_map(_per_set, mesh=mesh, in_specs=(P("s"), P("s")),
                        out_specs=P("s"), check_rep=False)
    sorted6 = sharded(rows, cols)

    def _final(s6):
        return pl.pallas_call(
            _mse_kernel,
            out_shape=jax.ShapeDtypeStruct((8, 128), jnp.float32),
        )(s6)

    loss = shard_map(_final, mesh=mesh, in_specs=P(), out_specs=P(),
                     check_rep=False)(sorted6)
    return loss[0, 0]


# speedup vs baseline: 47.9946x; 1.0194x over previous
import functools

import jax
import jax.numpy as jnp
from jax import lax
from jax.experimental import pallas as pl
from jax.experimental.pallas import tpu as pltpu
from jax.experimental.shard_map import shard_map
from jax.sharding import PartitionSpec as P

_N = 4096
_R = 512
_NN = 6
_KS = ((1, -1, 100), (3, 2, 20), (5, 4, 10))


def _knn_kernel(rows_ref, cols_ref, out_ref):
    i = pl.program_id(1)
    rx = rows_ref[0:1, :, 0:1].reshape(_R, 1)
    ry = rows_ref[0:1, :, 1:2].reshape(_R, 1)
    cx = cols_ref[0:1, 0:1, :].reshape(1, _N)
    cy = cols_ref[0:1, 1:2, :].reshape(1, _N)
    d2 = (rx - cx) ** 2 + (ry - cy) ** 2
    iota = lax.broadcasted_iota(jnp.int32, (_R, _N), 1)
    self_col = lax.broadcasted_iota(jnp.int32, (_R, _N), 0) + i * _R
    d2 = jnp.where(iota == self_col, jnp.inf, d2)

    m = [jnp.full((_R, 128), jnp.inf, dtype=jnp.float32) for _ in range(_NN)]
    for c in range(_N // 128):
        new = d2[:, c * 128:(c + 1) * 128]
        for j in range(_NN):
            lo = jnp.minimum(m[j], new)
            new = jnp.maximum(m[j], new)
            m[j] = lo

    cat = jnp.concatenate(m, axis=1)
    w = cat.shape[1]
    miota = lax.broadcasted_iota(jnp.int32, (_R, w), 1)
    for k in range(1, _NN):
        mn = jnp.min(cat, axis=1, keepdims=True)
        out_ref[0:1, :, k:k + 1] = jnp.sqrt(mn + 1e-12).reshape(1, _R, 1)
        if k + 1 < _NN:
            idx = jnp.min(jnp.where(cat == mn, miota, w), axis=1, keepdims=True)
            cat = jnp.where(miota == idx, jnp.inf, cat)


def _bitonic_sort_asc(x, flat):
    for k in range(1, 13):
        kbit = 1 << k
        asc = (flat & kbit) == 0
        for j in range(k - 1, -1, -1):
            s = 1 << j
            if s < 128:
                pm = pltpu.roll(x, 128 - s, 2)
                pp = pltpu.roll(x, s, 2)
            else:
                r = s // 128
                pm = pltpu.roll(x, 32 - r, 1)
                pp = pltpu.roll(x, r, 1)
            low = (flat & s) == 0
            p = jnp.where(low[None], pm, pp)
            keep_min = (low == asc)[None]
            x = jnp.where(keep_min, jnp.minimum(x, p), jnp.maximum(x, p))
    return x


def _sort_kernel(nn_ref, out_ref):
    flat = (lax.broadcasted_iota(jnp.int32, (32, 128), 0) * 128
            + lax.broadcasted_iota(jnp.int32, (32, 128), 1))

    def lengths(hi, lo):
        v = nn_ref[0:1, :, hi:hi + 1].reshape(32, 128)
        if lo >= 0:
            v = v - nn_ref[0:1, :, lo:lo + 1].reshape(32, 128)
        return v

    arrs = [lengths(hi, lo) for hi, lo, _ in _KS]
    out_ref[...] = _bitonic_sort_asc(jnp.stack(arrs), flat).reshape(1, 3, 32, 128)


def _mse_kernel(s_ref, out_ref):
    flat = (lax.broadcasted_iota(jnp.int32, (32, 128), 0) * 128
            + lax.broadcasted_iota(jnp.int32, (32, 128), 1))
    loss = jnp.float32(0.0)
    for i, (hi, lo, k) in enumerate(_KS):
        w = jnp.where(flat >= _N - k, jnp.float32(1.0 / k), jnp.float32(0.0))
        d = (s_ref[0:1, i:i + 1].reshape(32, 128)
             - s_ref[1:2, i:i + 1].reshape(32, 128))
        loss = loss + jnp.sum(d * d * w)
    out_ref[...] = jnp.full((8, 128), loss, dtype=jnp.float32)


def _per_set(rows_l, cols_l):
    nn = pl.pallas_call(
        _knn_kernel,
        grid=(1, _N // _R),
        in_specs=[
            pl.BlockSpec((1, _R, 2), lambda s, i: (s, i, 0)),
            pl.BlockSpec((1, 2, _N), lambda s, i: (s, 0, 0)),
        ],
        out_specs=pl.BlockSpec((1, _R, 8), lambda s, i: (s, i, 0)),
        out_shape=jax.ShapeDtypeStruct((1, _N, 8), jnp.float32),
        compiler_params=pltpu.CompilerParams(
            dimension_semantics=("parallel", "parallel")),
    )(rows_l, cols_l)

    sorted3 = pl.pallas_call(
        _sort_kernel,
        out_shape=jax.ShapeDtypeStruct((1, 3, 32, 128), jnp.float32),
    )(nn)

    both = jax.lax.all_gather(sorted3, "s", axis=0, tiled=True)
    return pl.pallas_call(
        _mse_kernel,
        out_shape=jax.ShapeDtypeStruct((8, 128), jnp.float32),
    )(both)


def kernel(x1, x2):
    c, h, w = x1.shape
    p1 = jnp.transpose(x1, (1, 2, 0)).reshape(-1, c)
    p2 = jnp.transpose(x2, (1, 2, 0)).reshape(-1, c)
    rows = jnp.stack([p1, p2])
    cols = jnp.stack([p1.T, p2.T])

    mesh = jax.make_mesh((2,), ("s",),
                         axis_types=(jax.sharding.AxisType.Auto,))
    sharded = shard_map(_per_set, mesh=mesh, in_specs=(P("s"), P("s")),
                        out_specs=P(), check_rep=False)
    loss = sharded(rows, cols)
    return loss[0, 0]


# speedup vs baseline: 68.3696x; 1.4245x over previous
import jax
import jax.numpy as jnp
from jax import lax
from jax.experimental import pallas as pl
from jax.experimental.pallas import tpu as pltpu

_N = 4096
_R = 512
_NN = 6
_KS = ((1, -1, 100), (3, 2, 20), (5, 4, 10))


def _oem(a, b, keep):
    n = len(a)
    if n == 1:
        out = [jnp.minimum(a[0], b[0])]
        if keep > 1:
            out.append(jnp.maximum(a[0], b[0]))
        return out
    ev = _oem(a[0::2], b[0::2], min(n, keep // 2 + 1))
    od = _oem(a[1::2], b[1::2], min(n, (keep + 1) // 2))
    out = [ev[0]]
    i = 0
    while len(out) < keep and i < len(od):
        if i + 1 < len(ev):
            out.append(jnp.minimum(od[i], ev[i + 1]))
            if len(out) < keep:
                out.append(jnp.maximum(od[i], ev[i + 1]))
        else:
            out.append(od[i])
        i += 1
    return out[:keep]


def _knn_kernel(rows_ref, cols_ref, out_ref):
    i = pl.program_id(1)
    rx = rows_ref[0:1, :, 0:1].reshape(_R, 1)
    ry = rows_ref[0:1, :, 1:2].reshape(_R, 1)
    cx = cols_ref[0:1, 0:1, :].reshape(1, _N)
    cy = cols_ref[0:1, 1:2, :].reshape(1, _N)
    d2 = (rx - cx) ** 2 + (ry - cy) ** 2
    iota = lax.broadcasted_iota(jnp.int32, (_R, _N), 1)
    self_col = lax.broadcasted_iota(jnp.int32, (_R, _N), 0) + i * _R
    d2 = jnp.where(iota == self_col, jnp.inf, d2)

    lists = [[d2[:, c * 128:(c + 1) * 128]] for c in range(_N // 128)]
    while len(lists) > 1:
        lists = [_oem(lists[2 * t], lists[2 * t + 1],
                      min(_NN, 2 * len(lists[2 * t])))
                 for t in range(len(lists) // 2)]
    m = lists[0]

    cat = jnp.concatenate(m, axis=1)
    w = cat.shape[1]
    miota = lax.broadcasted_iota(jnp.int32, (_R, w), 1)
    for k in range(1, _NN):
        mn = jnp.min(cat, axis=1, keepdims=True)
        out_ref[0:1, :, k:k + 1] = jnp.sqrt(mn + 1e-12).reshape(1, _R, 1)
        if k + 1 < _NN:
            idx = jnp.min(jnp.where(cat == mn, miota, w), axis=1, keepdims=True)
            cat = jnp.where(miota == idx, jnp.inf, cat)


def _bitonic_sort_asc(x, flat):
    for k in range(1, 13):
        kbit = 1 << k
        asc = (flat & kbit) == 0
        for j in range(k - 1, -1, -1):
            s = 1 << j
            if s < 128:
                pm = pltpu.roll(x, 128 - s, 2)
                pp = pltpu.roll(x, s, 2)
            else:
                r = s // 128
                pm = pltpu.roll(x, 32 - r, 1)
                pp = pltpu.roll(x, r, 1)
            low = (flat & s) == 0
            p = jnp.where(low[None], pm, pp)
            keep_min = (low == asc)[None]
            x = jnp.where(keep_min, jnp.minimum(x, p), jnp.maximum(x, p))
    return x


def _loss_kernel(nn_ref, out_ref):
    flat = (lax.broadcasted_iota(jnp.int32, (32, 128), 0) * 128
            + lax.broadcasted_iota(jnp.int32, (32, 128), 1))

    def lengths(s, hi, lo):
        v = nn_ref[s:s + 1, :, hi:hi + 1].reshape(32, 128)
        if lo >= 0:
            v = v - nn_ref[s:s + 1, :, lo:lo + 1].reshape(32, 128)
        return v

    arrs = []
    for hi, lo, k in _KS:
        arrs.append(lengths(0, hi, lo))
        arrs.append(lengths(1, hi, lo))
    x = _bitonic_sort_asc(jnp.stack(arrs), flat)
    loss = jnp.float32(0.0)
    for i, (hi, lo, k) in enumerate(_KS):
        w = jnp.where(flat >= _N - k, jnp.float32(1.0 / k), jnp.float32(0.0))
        d = x[2 * i] - x[2 * i + 1]
        loss = loss + jnp.sum(d * d * w)
    out_ref[...] = jnp.full((8, 128), loss, dtype=jnp.float32)


def kernel(x1, x2):
    c, h, w = x1.shape
    p1 = jnp.transpose(x1, (1, 2, 0)).reshape(-1, c)
    p2 = jnp.transpose(x2, (1, 2, 0)).reshape(-1, c)
    rows = jnp.stack([p1, p2])
    cols = jnp.stack([p1.T, p2.T])

    nn = pl.pallas_call(
        _knn_kernel,
        grid=(2, _N // _R),
        in_specs=[
            pl.BlockSpec((1, _R, 2), lambda s, i: (s, i, 0)),
            pl.BlockSpec((1, 2, _N), lambda s, i: (s, 0, 0)),
        ],
        out_specs=pl.BlockSpec((1, _R, 8), lambda s, i: (s, i, 0)),
        out_shape=jax.ShapeDtypeStruct((2, _N, 8), jnp.float32),
        compiler_params=pltpu.CompilerParams(
            dimension_semantics=("parallel", "parallel")),
    )(rows, cols)

    loss = pl.pallas_call(
        _loss_kernel,
        out_shape=jax.ShapeDtypeStruct((8, 128), jnp.float32),
    )(nn)
    return loss[0, 0]


# speedup vs baseline: 79.7350x; 1.1662x over previous
import jax
import jax.numpy as jnp
from jax import lax
from jax.experimental import pallas as pl
from jax.experimental.pallas import tpu as pltpu

_N = 4096
_R = 512
_NN = 6
_KS = ((1, -1, 100), (3, 2, 20), (5, 4, 10))


def _oem(a, b, keep):
    n = len(a)
    if n == 1:
        out = [jnp.minimum(a[0], b[0])]
        if keep > 1:
            out.append(jnp.maximum(a[0], b[0]))
        return out
    ev = _oem(a[0::2], b[0::2], min(n, keep // 2 + 1))
    od = _oem(a[1::2], b[1::2], min(n, (keep + 1) // 2))
    out = [ev[0]]
    i = 0
    while len(out) < keep and i < len(od):
        if i + 1 < len(ev):
            out.append(jnp.minimum(od[i], ev[i + 1]))
            if len(out) < keep:
                out.append(jnp.maximum(od[i], ev[i + 1]))
        else:
            out.append(od[i])
        i += 1
    return out[:keep]


def _knn_kernel(rows_ref, cols_ref, out_ref):
    rx = rows_ref[0:1, :, 0:1].reshape(_R, 1)
    ry = rows_ref[0:1, :, 1:2].reshape(_R, 1)
    cx = cols_ref[0:1, 0:1, :].reshape(1, _N)
    cy = cols_ref[0:1, 1:2, :].reshape(1, _N)
    d2 = (rx - cx) ** 2 + (ry - cy) ** 2

    lists = [[d2[:, c * 128:(c + 1) * 128]] for c in range(_N // 128)]
    keep = _NN
    while len(lists) > 1:
        lists = [_oem(lists[2 * t], lists[2 * t + 1],
                      min(keep, 2 * len(lists[2 * t])))
                 for t in range(len(lists) // 2)]
    m = lists[0]

    liota = lax.broadcasted_iota(jnp.int32, (_R, 128), 1)
    for k in range(_NN):
        mn = jnp.min(m[0], axis=1, keepdims=True)
        if k > 0:
            out_ref[0:1, :, k:k + 1] = jnp.sqrt(mn + 1e-12).reshape(1, _R, 1)
        if k + 1 < _NN:
            lane = jnp.min(jnp.where(m[0] == mn, liota, 128),
                           axis=1, keepdims=True)
            shift = liota == lane
            for j in range(keep - 1):
                m[j] = jnp.where(shift, m[j + 1], m[j])


def _bitonic_sort_asc(x, flat):
    for k in range(1, 13):
        kbit = 1 << k
        asc = (flat & kbit) == 0
        for j in range(k - 1, -1, -1):
            s = 1 << j
            if s < 128:
                pm = pltpu.roll(x, 128 - s, 2)
                pp = pltpu.roll(x, s, 2)
            else:
                r = s // 128
                pm = pltpu.roll(x, 32 - r, 1)
                pp = pltpu.roll(x, r, 1)
            low = (flat & s) == 0
            p = jnp.where(low[None], pm, pp)
            keep_min = (low == asc)[None]
            x = jnp.where(keep_min, jnp.minimum(x, p), jnp.maximum(x, p))
    return x


def _loss_kernel(nn_ref, out_ref):
    flat = (lax.broadcasted_iota(jnp.int32, (32, 128), 0) * 128
            + lax.broadcasted_iota(jnp.int32, (32, 128), 1))

    def lengths(s, hi, lo):
        v = nn_ref[s:s + 1, :, hi:hi + 1].reshape(32, 128)
        if lo >= 0:
            v = v - nn_ref[s:s + 1, :, lo:lo + 1].reshape(32, 128)
        return v

    arrs = []
    for hi, lo, k in _KS:
        arrs.append(lengths(0, hi, lo))
        arrs.append(lengths(1, hi, lo))
    x = _bitonic_sort_asc(jnp.stack(arrs), flat)
    loss = jnp.float32(0.0)
    for i, (hi, lo, k) in enumerate(_KS):
        w = jnp.where(flat >= _N - k, jnp.float32(1.0 / k), jnp.float32(0.0))
        d = x[2 * i] - x[2 * i + 1]
        loss = loss + jnp.sum(d * d * w)
    out_ref[...] = jnp.full((8, 128), loss, dtype=jnp.float32)


def kernel(x1, x2):
    c, h, w = x1.shape
    p1 = jnp.transpose(x1, (1, 2, 0)).reshape(-1, c)
    p2 = jnp.transpose(x2, (1, 2, 0)).reshape(-1, c)
    rows = jnp.stack([p1, p2])
    cols = jnp.stack([p1.T, p2.T])

    nn = pl.pallas_call(
        _knn_kernel,
        grid=(2, _N // _R),
        in_specs=[
            pl.BlockSpec((1, _R, 2), lambda s, i: (s, i, 0)),
            pl.BlockSpec((1, 2, _N), lambda s, i: (s, 0, 0)),
        ],
        out_specs=pl.BlockSpec((1, _R, 8), lambda s, i: (s, i, 0)),
        out_shape=jax.ShapeDtypeStruct((2, _N, 8), jnp.float32),
        compiler_params=pltpu.CompilerParams(
            dimension_semantics=("parallel", "parallel")),
    )(rows, cols)

    loss = pl.pallas_call(
        _loss_kernel,
        out_shape=jax.ShapeDtypeStruct((8, 128), jnp.float32),
    )(nn)
    return loss[0, 0]


# speedup vs baseline: 80.6562x; 1.0116x over previous
import jax
import jax.numpy as jnp
from jax import lax
from jax.experimental import pallas as pl
from jax.experimental.pallas import tpu as pltpu

_N = 4096
_R = 512
_NN = 6
_KS = ((1, -1, 100), (3, 2, 20), (5, 4, 10))


def _oem(a, b, keep):
    n = len(a)
    if n == 1:
        out = [jnp.minimum(a[0], b[0])]
        if keep > 1:
            out.append(jnp.maximum(a[0], b[0]))
        return out
    ev = _oem(a[0::2], b[0::2], min(n, keep // 2 + 1))
    od = _oem(a[1::2], b[1::2], min(n, (keep + 1) // 2))
    out = [ev[0]]
    i = 0
    while len(out) < keep and i < len(od):
        if i + 1 < len(ev):
            out.append(jnp.minimum(od[i], ev[i + 1]))
            if len(out) < keep:
                out.append(jnp.maximum(od[i], ev[i + 1]))
        else:
            out.append(od[i])
        i += 1
    return out[:keep]


def _knn_kernel(rows_ref, cols_ref, out_ref):
    rx = rows_ref[0:1, :, 0:1].reshape(_R, 1)
    ry = rows_ref[0:1, :, 1:2].reshape(_R, 1)
    cx = cols_ref[0:1, 0:1, :].reshape(1, _N)
    cy = cols_ref[0:1, 1:2, :].reshape(1, _N)
    d2 = (rx - cx) ** 2 + (ry - cy) ** 2

    lists = [[d2[:, c * 128:(c + 1) * 128]] for c in range(_N // 128)]
    keep = _NN
    while len(lists) > 1:
        lists = [_oem(lists[2 * t], lists[2 * t + 1],
                      min(keep, 2 * len(lists[2 * t])))
                 for t in range(len(lists) // 2)]
    m = lists[0]

    liota = lax.broadcasted_iota(jnp.int32, (_R, 128), 1)
    for k in range(_NN):
        mn = jnp.min(m[0], axis=1, keepdims=True)
        if k > 0:
            out_ref[0:1, :, k:k + 1] = jnp.sqrt(mn + 1e-12).reshape(1, _R, 1)
        if k + 1 < _NN:
            lane = jnp.min(jnp.where(m[0] == mn, liota, 128),
                           axis=1, keepdims=True)
            shift = liota == lane
            for j in range(keep - 1):
                m[j] = jnp.where(shift, m[j + 1], m[j])


def _lane_stage(x, lane, s, keep_min):
    pm = pltpu.roll(x, 128 - s, 2)
    pp = pltpu.roll(x, s, 2)
    low = (lane & s) == 0
    p = jnp.where(low[None], pm, pp)
    return jnp.where(keep_min[None], jnp.minimum(x, p), jnp.maximum(x, p))


def _row_dirs(rows):
    ri = lax.broadcasted_iota(jnp.int32, (rows, 1), 0)
    return ri < max(1, rows // 2)


def _top128_asc(x, lane):
    rows = x.shape[1]
    for k in range(1, 7):
        asc = (lane & (1 << k)) == 0
        for j in range(k - 1, -1, -1):
            s = 1 << j
            low = (lane & s) == 0
            x = _lane_stage(x, lane, s, low == asc)
    d = _row_dirs(rows)
    for j in range(6, -1, -1):
        s = 1 << j
        x = _lane_stage(x, lane, s, ((lane & s) == 0) == d)
    while rows > 1:
        h = rows // 2
        x = jnp.maximum(x[:, :h, :], x[:, h:, :])
        rows = h
        d = _row_dirs(rows)
        for j in range(6, -1, -1):
            s = 1 << j
            x = _lane_stage(x, lane, s, ((lane & s) == 0) == d)
    return x


def _loss_kernel(nn_ref, out_ref):
    lane = lax.broadcasted_iota(jnp.int32, (1, 128), 1)

    def lengths(s, hi, lo):
        v = nn_ref[s:s + 1, :, hi:hi + 1].reshape(32, 128)
        if lo >= 0:
            v = v - nn_ref[s:s + 1, :, lo:lo + 1].reshape(32, 128)
        return v

    arrs = []
    for hi, lo, k in _KS:
        arrs.append(lengths(0, hi, lo))
        arrs.append(lengths(1, hi, lo))
    x = _top128_asc(jnp.stack(arrs), lane)
    loss = jnp.float32(0.0)
    for i, (hi, lo, k) in enumerate(_KS):
        w = jnp.where(lane >= 128 - k, jnp.float32(1.0 / k), jnp.float32(0.0))
        d = x[2 * i] - x[2 * i + 1]
        loss = loss + jnp.sum(d * d * w)
    out_ref[...] = jnp.full((8, 128), loss, dtype=jnp.float32)


def kernel(x1, x2):
    c, h, w = x1.shape
    p1 = jnp.transpose(x1, (1, 2, 0)).reshape(-1, c)
    p2 = jnp.transpose(x2, (1, 2, 0)).reshape(-1, c)
    rows = jnp.stack([p1, p2])
    cols = jnp.stack([p1.T, p2.T])

    nn = pl.pallas_call(
        _knn_kernel,
        grid=(2, _N // _R),
        in_specs=[
            pl.BlockSpec((1, _R, 2), lambda s, i: (s, i, 0)),
            pl.BlockSpec((1, 2, _N), lambda s, i: (s, 0, 0)),
        ],
        out_specs=pl.BlockSpec((1, _R, 8), lambda s, i: (s, i, 0)),
        out_shape=jax.ShapeDtypeStruct((2, _N, 8), jnp.float32),
        compiler_params=pltpu.CompilerParams(
            dimension_semantics=("parallel", "parallel")),
    )(rows, cols)

    loss = pl.pallas_call(
        _loss_kernel,
        out_shape=jax.ShapeDtypeStruct((8, 128), jnp.float32),
    )(nn)
    return loss[0, 0]
